# baseline (speedup 1.0000x reference)
import jax
import jax.numpy as jnp
from jax.experimental import pallas as pl

_BC = 8192


def _body(s_ref, m_ref, o_ref):
    s = s_ref[...]
    l = pl.program_id(1)
    mi = m_ref[...].astype(jnp.int32)
    ml = jnp.where(l == 0, mi[:, 0:1],
                   jnp.where(l == 1, mi[:, 1:2], mi[:, 2:3]))
    o_ref[0] = jnp.where(ml != 0, jnp.float32(-jnp.inf), s)


def kernel(scores, M):
    B, C = scores.shape
    L = M.shape[0]
    sT = jnp.swapaxes(scores, 0, 1)
    mT = jnp.swapaxes(M, 0, 1)
    outT = pl.pallas_call(
        _body,
        grid=(B // _BC, L),
        in_specs=[
            pl.BlockSpec((C, _BC), lambda j, l: (0, j)),
            pl.BlockSpec((C, L), lambda j, l: (0, 0)),
        ],
        out_specs=pl.BlockSpec((1, C, _BC), lambda j, l: (l, 0, j)),
        out_shape=jax.ShapeDtypeStruct((L, C, B), scores.dtype),
    )(sT, mT)
    return jnp.transpose(outT, (2, 0, 1))
